# baseline (speedup 1.0000x reference)
import functools

import jax
import jax.numpy as jnp
from jax import lax
from jax.experimental import pallas as pl
from jax.experimental.pallas import tpu as pltpu
from jax.experimental.pallas import tpu_sc as plsc

B = 16384
D = 64
L = 16

_info = plsc.get_sparse_core_info()
NC, NS = _info.num_cores, _info.num_subcores
NW = NC * NS
BPW = B // NW
CH = 64
NCHUNK = BPW // CH


def _sc_body(idx_hbm, w_hbm, v_hbm, u_hbm, out_hbm,
             idx_v, q_v, wgA, vgA, ugA, wgB, vgB, ugB, out_v, semA, semB):
    wid = lax.axis_index("s") * NC + lax.axis_index("c")

    pltpu.sync_copy(idx_hbm.at[wid], idx_v)

    for t in range(3):
        for ci in range(NCHUNK):
            for g in range(CH // L):
                src = idx_v[pl.ds(t * BPW + ci * CH + g * L, L)]
                q_v[t, ci, pl.ds(g * L, L)] = src >> 2

    lane = jnp.arange(L, dtype=jnp.int32)
    tabs = (w_hbm, v_hbm, u_hbm)
    bufsA = (wgA, vgA, ugA)
    bufsB = (wgB, vgB, ugB)

    def fire(ci, bufs, sem):
        for t in range(3):
            pltpu.async_copy(tabs[t].at[q_v.at[t, ci]], bufs[t], sem)

    def drain(bufs, sem):
        for t in range(3):
            pltpu.make_async_copy(tabs[t].at[q_v.at[0, 0]], bufs[t],
                                  sem).wait()

    def compute(ci, bufs):
        wg, vg, ug = bufs
        for g in range(CH // L):
            ivecs = [idx_v[pl.ds(t * BPW + ci * CH + g * L, L)]
                     for t in range(3)]
            offs = [(iv & 3) << 5 for iv in ivecs]
            thetas = jnp.zeros((L,), jnp.float32)
            for r in range(L):
                j = g * L + r
                ow, ov, ou = offs[0][r], offs[1][r], offs[2][r]
                acc = jnp.zeros((L,), jnp.float32)
                for c in range(2):
                    wb = plsc.bitcast(wg[j, pl.ds(ow + c * L, L)],
                                      jnp.bfloat16)
                    vb = plsc.bitcast(vg[j, pl.ds(ov + c * L, L)],
                                      jnp.bfloat16)
                    ub = plsc.bitcast(ug[j, pl.ds(ou + c * L, L)],
                                      jnp.bfloat16)
                    wv = plsc.unpack(wb, format=plsc.PackFormat.INTERLEAVED,
                                     preferred_element_type=jnp.float32)
                    vv = plsc.unpack(vb, format=plsc.PackFormat.INTERLEAVED,
                                     preferred_element_type=jnp.float32)
                    uv = plsc.unpack(ub, format=plsc.PackFormat.INTERLEAVED,
                                     preferred_element_type=jnp.float32)
                    acc = acc + wv[0] * vv[0] * uv[0] + wv[1] * vv[1] * uv[1]
                theta = jnp.sum(acc)
                thetas = thetas + jnp.where(lane == r, theta, 0.0)
            probs = 1.0 / (1.0 + jnp.exp(-thetas))
            out_v[pl.ds(ci * CH + g * L, L)] = probs

    fire(0, bufsA, semA)
    fire(1, bufsB, semB)

    def body(m, carry):
        c0 = 2 * m
        drain(bufsA, semA)
        compute(c0, bufsA)
        fire(c0 + 2, bufsA, semA)
        drain(bufsB, semB)
        compute(c0 + 1, bufsB)
        fire(c0 + 3, bufsB, semB)
        return carry

    lax.fori_loop(0, NCHUNK // 2 - 1, body, 0)

    drain(bufsA, semA)
    compute(NCHUNK - 2, bufsA)
    drain(bufsB, semB)
    compute(NCHUNK - 1, bufsB)

    pltpu.sync_copy(out_v, out_hbm.at[pl.ds(wid * BPW, BPW)])


@functools.partial(jax.jit, static_argnums=())
def kernel(indices, W, V, U):
    def _pack(T):
        b3 = T.astype(jnp.bfloat16).reshape(T.shape[0] // 4, 2 * D, 2)
        return jax.lax.bitcast_convert_type(b3, jnp.int32)

    Wl, Vl, Ul = _pack(W), _pack(V), _pack(U)
    idx = indices.astype(jnp.int32).T
    idx = idx.reshape(3, NW, BPW).transpose(1, 0, 2).reshape(NW, 3 * BPW)

    mesh = plsc.VectorSubcoreMesh(core_axis_name="c", subcore_axis_name="s")
    run = pl.kernel(
        _sc_body,
        mesh=mesh,
        out_type=jax.ShapeDtypeStruct((B,), jnp.float32),
        scratch_types=[
            pltpu.VMEM((3 * BPW,), jnp.int32),
            pltpu.VMEM((3, NCHUNK, CH), jnp.int32),
            pltpu.VMEM((CH, 2 * D), jnp.int32),
            pltpu.VMEM((CH, 2 * D), jnp.int32),
            pltpu.VMEM((CH, 2 * D), jnp.int32),
            pltpu.VMEM((CH, 2 * D), jnp.int32),
            pltpu.VMEM((CH, 2 * D), jnp.int32),
            pltpu.VMEM((CH, 2 * D), jnp.int32),
            pltpu.VMEM((BPW,), jnp.float32),
            pltpu.SemaphoreType.DMA,
            pltpu.SemaphoreType.DMA,
        ],
        compiler_params=pltpu.CompilerParams(needs_layout_passes=False),
    )
    return run(idx, Wl, Vl, Ul)


# speedup vs baseline: 22.7029x; 22.7029x over previous
import functools

import jax
import jax.numpy as jnp
from jax import lax
from jax.experimental import pallas as pl
from jax.experimental.pallas import tpu as pltpu
from jax.experimental.pallas import tpu_sc as plsc

B = 16384
D = 64
L = 16

_info = plsc.get_sparse_core_info()
NC, NS = _info.num_cores, _info.num_subcores
NW = NC * NS
BPW = B // NW
CH = 64
NCHUNK = BPW // CH

_PACK_ROWS = 2000


def _pack_body(in_ref, out_ref):
    x = in_ref[...]
    out_ref[:, 0:D] = x
    out_ref[:, D:2 * D] = x


def _pack_table(T):
    n = T.shape[0]
    return pl.pallas_call(
        _pack_body,
        grid=(n // _PACK_ROWS,),
        in_specs=[pl.BlockSpec((_PACK_ROWS, D), lambda i: (i, 0))],
        out_specs=pl.BlockSpec((_PACK_ROWS, 2 * D), lambda i: (i, 0)),
        out_shape=jax.ShapeDtypeStruct((n, 2 * D), jnp.float32),
    )(T)


def _sc_body(idx_hbm, w_hbm, v_hbm, u_hbm, out_hbm,
             idx_v, q_v, wgA, vgA, ugA, wgB, vgB, ugB, out_v, semA, semB):
    wid = lax.axis_index("s") * NC + lax.axis_index("c")

    pltpu.sync_copy(idx_hbm.at[wid], idx_v)

    for t in range(3):
        for ci in range(NCHUNK):
            for g in range(CH // L):
                src = idx_v[pl.ds(t * BPW + ci * CH + g * L, L)]
                q_v[t, ci, pl.ds(g * L, L)] = src

    lane = jnp.arange(L, dtype=jnp.int32)
    tabs = (w_hbm, v_hbm, u_hbm)
    bufsA = (wgA, vgA, ugA)
    bufsB = (wgB, vgB, ugB)

    def fire(ci, bufs, sem):
        for t in range(3):
            pltpu.async_copy(tabs[t].at[q_v.at[t, ci]], bufs[t], sem)

    def drain(bufs, sem):
        for t in range(3):
            pltpu.make_async_copy(tabs[t].at[q_v.at[0, 0]], bufs[t],
                                  sem).wait()

    def compute(ci, bufs):
        wg, vg, ug = bufs
        for g in range(CH // L):
            thetas = jnp.zeros((L,), jnp.float32)
            for r in range(L):
                j = g * L + r
                acc = jnp.zeros((L,), jnp.float32)
                for c in range(D // L):
                    sl = pl.ds(c * L, L)
                    acc = acc + wg[j, sl] * vg[j, sl] * ug[j, sl]
                theta = jnp.sum(acc)
                thetas = thetas + jnp.where(lane == r, theta, 0.0)
            probs = 1.0 / (1.0 + jnp.exp(-thetas))
            out_v[pl.ds(ci * CH + g * L, L)] = probs

    fire(0, bufsA, semA)
    fire(1, bufsB, semB)

    def body(m, carry):
        c0 = 2 * m
        drain(bufsA, semA)
        compute(c0, bufsA)
        fire(c0 + 2, bufsA, semA)
        drain(bufsB, semB)
        compute(c0 + 1, bufsB)
        fire(c0 + 3, bufsB, semB)
        return carry

    lax.fori_loop(0, NCHUNK // 2 - 1, body, 0)

    drain(bufsA, semA)
    compute(NCHUNK - 2, bufsA)
    drain(bufsB, semB)
    compute(NCHUNK - 1, bufsB)

    pltpu.sync_copy(out_v, out_hbm.at[pl.ds(wid * BPW, BPW)])


@functools.partial(jax.jit, static_argnums=())
def kernel(indices, W, V, U):
    Wl = _pack_table(W)
    Vl = _pack_table(V)
    Ul = _pack_table(U)
    idx = indices.astype(jnp.int32).T
    idx = idx.reshape(3, NW, BPW).transpose(1, 0, 2).reshape(NW, 3 * BPW)

    mesh = plsc.VectorSubcoreMesh(core_axis_name="c", subcore_axis_name="s")
    run = pl.kernel(
        _sc_body,
        mesh=mesh,
        out_type=jax.ShapeDtypeStruct((B,), jnp.float32),
        scratch_types=[
            pltpu.VMEM((3 * BPW,), jnp.int32),
            pltpu.VMEM((3, NCHUNK, CH), jnp.int32),
            pltpu.VMEM((CH, 2 * D), jnp.float32),
            pltpu.VMEM((CH, 2 * D), jnp.float32),
            pltpu.VMEM((CH, 2 * D), jnp.float32),
            pltpu.VMEM((CH, 2 * D), jnp.float32),
            pltpu.VMEM((CH, 2 * D), jnp.float32),
            pltpu.VMEM((CH, 2 * D), jnp.float32),
            pltpu.VMEM((BPW,), jnp.float32),
            pltpu.SemaphoreType.DMA,
            pltpu.SemaphoreType.DMA,
        ],
        compiler_params=pltpu.CompilerParams(needs_layout_passes=False),
    )
    return run(idx, Wl, Vl, Ul)


# speedup vs baseline: 36.2938x; 1.5986x over previous
import functools

import jax
import jax.numpy as jnp
from jax import lax
from jax.experimental import pallas as pl
from jax.experimental.pallas import tpu as pltpu
from jax.experimental.pallas import tpu_sc as plsc

B = 16384
D = 64
L = 16

_info = plsc.get_sparse_core_info()
NC, NS = _info.num_cores, _info.num_subcores
NW = NC * NS
BPW = B // NW
NCHUNK = 4
CHUNK = BPW // NCHUNK
NBLK = BPW // L


def _sc_body(idx_hbm, w_hbm, v_hbm, u_hbm, out_hbm,
             idx_v, w_rows, v_rows, u_rows, out_v, sem):
    wid = lax.axis_index("s") * NC + lax.axis_index("c")

    pltpu.sync_copy(idx_hbm.at[wid], idx_v)

    handles = []
    for t, (tab, rows) in enumerate(
            ((w_hbm, w_rows), (v_hbm, v_rows), (u_hbm, u_rows))):
        for c in range(NCHUNK):
            handles.append(pltpu.async_copy(
                tab.at[idx_v.at[t, c]],
                rows.at[pl.ds(c * CHUNK, CHUNK), :],
                sem))
    for h in handles:
        h.wait()

    lane = jnp.arange(L, dtype=jnp.int32)

    def blk_body(blk, carry):
        base = blk * L
        thetas = jnp.zeros((L,), jnp.float32)
        for r in range(L):
            row = base + r
            acc = jnp.zeros((L,), jnp.float32)
            for c in range(D // L):
                sl = pl.ds(c * L, L)
                acc = acc + w_rows[row, sl] * v_rows[row, sl] * u_rows[row, sl]
            theta = jnp.sum(acc)
            thetas = thetas + jnp.where(lane == r, theta, 0.0)
        probs = 1.0 / (1.0 + jnp.exp(-thetas))
        out_v[pl.ds(base, L)] = probs
        return carry

    lax.fori_loop(0, NBLK, blk_body, 0)

    pltpu.sync_copy(out_v, out_hbm.at[pl.ds(wid * BPW, BPW)])


@functools.partial(jax.jit, static_argnums=())
def kernel(indices, W, V, U):
    idx = indices.astype(jnp.int32).T
    idx = idx.reshape(3, NW, NCHUNK, CHUNK).transpose(1, 0, 2, 3)

    mesh = plsc.VectorSubcoreMesh(core_axis_name="c", subcore_axis_name="s")
    run = pl.kernel(
        _sc_body,
        mesh=mesh,
        out_type=jax.ShapeDtypeStruct((B,), jnp.float32),
        scratch_types=[
            pltpu.VMEM((3, NCHUNK, CHUNK), jnp.int32),
            pltpu.VMEM((BPW, D), jnp.float32),
            pltpu.VMEM((BPW, D), jnp.float32),
            pltpu.VMEM((BPW, D), jnp.float32),
            pltpu.VMEM((BPW,), jnp.float32),
            pltpu.SemaphoreType.DMA,
        ],
        compiler_params=pltpu.CompilerParams(
            needs_layout_passes=False, use_tc_tiling_on_sc=False),
    )
    return run(idx, W, V, U)
